# baseline (speedup 1.0000x reference)
import jax
import jax.numpy as jnp
from jax.experimental import pallas as pl
from jax.experimental.pallas import tpu as pltpu


def _prelu(h, a):
    return jnp.maximum(h, 0.0) + a * jnp.minimum(h, 0.0)


def _dense(h, w_ref, b_ref):
    return (jnp.dot(h.astype(w_ref.dtype), w_ref[...],
                    preferred_element_type=jnp.float32)
            + b_ref[...])


def _net_block(*refs):
    x_ref = refs[0]
    al_ref = refs[1]
    z_ref, f_ref, r_ref = refs[-3:]
    nview = x_ref.shape[0]
    per = (len(refs) - 5) // nview
    for v in range(nview):
        (ew1, eb1, ew2, eb2, ew3, eb3, ew4, eb4,
         dw1, db1, dw2, db2, dw3, db3, dw4, db4,
         pw, pb) = refs[2 + v * per: 2 + (v + 1) * per]
        x = x_ref[v]
        al = al_ref[v, 0]

        h = _prelu(_dense(x, ew1, eb1), al[0])
        h = _prelu(_dense(h, ew2, eb2), al[1])
        h = _prelu(_dense(h, ew3, eb3), al[2])
        z = _dense(h, ew4, eb4)

        g = _prelu(_dense(z, dw1, db1), al[3])
        g = _prelu(_dense(g, dw2, db2), al[4])
        g = _prelu(_dense(g, dw3, db3), al[5])
        r = _dense(g, dw4, db4)

        f = _dense(z, pw, pb)

        z_ref[v] = z
        f_ref[v] = f
        r_ref[v] = r


_TILE_CANDIDATES = (1000, 400, 256, 200, 128, 80, 64, 40, 32, 16, 8)


def kernel(xs, enc_params, dec_params, proj_params):
    view, n, din = xs.shape
    nlayers = len(enc_params[0])
    tile = next(t for t in _TILE_CANDIDATES if n % t == 0)

    alphas = jnp.stack([
        jnp.concatenate([e[l][2] for l in range(nlayers - 1)]
                        + [d[l][2] for l in range(nlayers - 1)])
        for e, d in zip(enc_params, dec_params)
    ])[:, None, :]

    def const_spec(arr):
        shape = arr.shape
        return pl.BlockSpec(shape, lambda i: (0,) * len(shape))

    operands = []
    in_specs = [pl.BlockSpec((view, tile, din), lambda i: (0, i, 0)),
                const_spec(alphas)]
    per_view = []
    for v in range(view):
        ops = []
        for (w, b, _a) in enc_params[v]:
            ops += [w.astype(jnp.bfloat16), b.reshape(1, -1)]
        for (w, b, _a) in dec_params[v]:
            ops += [w.astype(jnp.bfloat16), b.reshape(1, -1)]
        pw, pb = proj_params[v]
        ops += [pw.astype(jnp.bfloat16), pb.reshape(1, -1)]
        per_view.append(ops)
    for ops in per_view:
        operands += ops
        in_specs += [const_spec(o) for o in ops]

    feat = enc_params[0][-1][0].shape[-1]
    high = proj_params[0][0].shape[-1]
    out_shape = (
        jax.ShapeDtypeStruct((view, n, feat), xs.dtype),
        jax.ShapeDtypeStruct((view, n, high), xs.dtype),
        jax.ShapeDtypeStruct((view, n, din), xs.dtype),
    )
    out_specs = (
        pl.BlockSpec((view, tile, feat), lambda i: (0, i, 0)),
        pl.BlockSpec((view, tile, high), lambda i: (0, i, 0)),
        pl.BlockSpec((view, tile, din), lambda i: (0, i, 0)),
    )

    return pl.pallas_call(
        _net_block,
        grid=(n // tile,),
        in_specs=in_specs,
        out_specs=out_specs,
        out_shape=out_shape,
        compiler_params=pltpu.CompilerParams(
            dimension_semantics=("arbitrary",),
            vmem_limit_bytes=100 * 1024 * 1024,
        ),
    )(xs, alphas, *operands)
